# Initial kernel scaffold; baseline (speedup 1.0000x reference)
import jax
import jax.numpy as jnp
from jax.experimental import pallas as pl


def kernel(gen_txt_seq, gen_bn_seq, gen_vis_seq, greedy_txt_seq, greedy_bn_seq, greedy_vis_seq, gt_gts, ncap, st2towidx, token_scores):
    raise NotImplementedError("write your pallas kernel here")


# speedup vs baseline: 52.1269x; 52.1269x over previous
import jax
import jax.numpy as jnp
from jax import lax
from jax.experimental import pallas as pl
from jax.experimental.pallas import tpu as pltpu
from jax.experimental.pallas import tpu_sc as plsc

VOCAB = 9487
BATCH = 4096
SEQ = 20
N_IMG = 1024
MAX_CAPS = 5

NW = 32
ROWS_W = BATCH // NW
IMGS_W = N_IMG // NW
SEQ_ELEMS_W = ROWS_W * SEQ
GT_ELEMS_W = IMGS_W * MAX_CAPS * SEQ
TOK_PAD = 9488
ST_PAD = 1024


def _sc_body(gtx, gbn, gvi, rtx, rbn, rvi, gt, ncap_h, st, tok,
             rew_out, ps_out,
             tok_v, st_v, gtx_v, gbn_v, gvi_v, rtx_v, rbn_v, rvi_v,
             gt_v, ncap_v, gimg_v, rew_v, acc_v, sem):
    wid = lax.axis_index("s") * 2 + lax.axis_index("c")
    sbase = wid * SEQ_ELEMS_W

    copies = [
        pltpu.async_copy(tok, tok_v, sem),
        pltpu.async_copy(st, st_v, sem),
        pltpu.async_copy(gtx.at[pl.ds(sbase, SEQ_ELEMS_W)], gtx_v, sem),
        pltpu.async_copy(gbn.at[pl.ds(sbase, SEQ_ELEMS_W)], gbn_v, sem),
        pltpu.async_copy(gvi.at[pl.ds(sbase, SEQ_ELEMS_W)], gvi_v, sem),
        pltpu.async_copy(rtx.at[pl.ds(sbase, SEQ_ELEMS_W)], rtx_v, sem),
        pltpu.async_copy(rbn.at[pl.ds(sbase, SEQ_ELEMS_W)], rbn_v, sem),
        pltpu.async_copy(rvi.at[pl.ds(sbase, SEQ_ELEMS_W)], rvi_v, sem),
        pltpu.async_copy(gt.at[pl.ds(wid * GT_ELEMS_W, GT_ELEMS_W)], gt_v, sem),
        pltpu.async_copy(ncap_h.at[pl.ds(wid * IMGS_W, IMGS_W)], ncap_v, sem),
    ]
    for c in copies:
        c.wait()

    iota = lax.iota(jnp.int32, 16)
    zero16 = jnp.zeros((16,), jnp.float32)

    for grp in range(IMGS_W // 16):
        ncap_i = ncap_v[pl.ds(grp * 16, 16)]
        ioff = iota * (MAX_CAPS * SEQ) + grp * 16 * (MAX_CAPS * SEQ)

        def cap_body(c, gsum):
            def t_body(t, carry):
                s, cnt = carry
                tid = plsc.load_gather(gt_v, [ioff + (c * SEQ + t)])
                ts = plsc.load_gather(tok_v, [tid])
                valid = tid != 0
                s = s + jnp.where(valid, ts, 0.0)
                cnt = cnt + jnp.where(valid, 1.0, 0.0)
                return s, cnt

            s, cnt = lax.fori_loop(0, SEQ, t_body, (zero16, zero16))
            cap_score = s / jnp.maximum(cnt, 1.0)
            return gsum + jnp.where(c < ncap_i, cap_score, 0.0)

        gsum = lax.fori_loop(0, MAX_CAPS, cap_body, zero16)
        gimg_v[pl.ds(grp * 16, 16)] = gsum / ncap_i.astype(jnp.float32)

    acc_v[...] = zero16

    @pl.loop(0, ROWS_W // 16)
    def _(g):
        rbase = g * 16 * SEQ

        def seq_score(txt_v, bn_v, vis_v):
            def t_body(t, carry):
                s, cnt = carry
                idx = iota * SEQ + rbase + t
                txt = plsc.load_gather(txt_v, [idx])
                bn = plsc.load_gather(bn_v, [idx])
                vis = plsc.load_gather(vis_v, [idx])
                mapped = plsc.load_gather(st_v, [vis * 2 + bn - 1])
                res = jnp.where(txt < VOCAB, txt, mapped)
                ts = plsc.load_gather(tok_v, [res])
                valid = res != 0
                s = s + jnp.where(valid, ts, 0.0)
                cnt = cnt + jnp.where(valid, 1.0, 0.0)
                return s, cnt

            s, cnt = lax.fori_loop(0, SEQ, t_body, (zero16, zero16))
            return s / jnp.maximum(cnt, 1.0)

        gen_s = seq_score(gtx_v, gbn_v, gvi_v)
        gre_s = seq_score(rtx_v, rbn_v, rvi_v)
        gtv = plsc.load_gather(gimg_v, [(g * 16 + iota) // 4])
        score = (gen_s - gre_s) * gtv
        acc_v[...] = acc_v[...] + score

        @pl.loop(0, SEQ)
        def _(t):
            plsc.store_scatter(rew_v, [iota * SEQ + rbase + t], score)

    pltpu.async_copy(rew_v, rew_out.at[pl.ds(sbase, SEQ_ELEMS_W)], sem).wait()
    pltpu.async_copy(acc_v, ps_out.at[wid], sem).wait()


def _mean_body(ps_ref, o_ref):
    o_ref[...] = jnp.full((1, 1), jnp.sum(ps_ref[...]) * (1.0 / BATCH),
                          jnp.float32)


@jax.jit
def kernel(gen_txt_seq, gen_bn_seq, gen_vis_seq, greedy_txt_seq,
           greedy_bn_seq, greedy_vis_seq, gt_gts, ncap, st2towidx,
           token_scores):
    i32 = jnp.int32
    gtx = gen_txt_seq.astype(i32).reshape(-1)
    gbn = gen_bn_seq.astype(i32).reshape(-1)
    gvi = gen_vis_seq.astype(i32).reshape(-1)
    rtx = greedy_txt_seq.astype(i32).reshape(-1)
    rbn = greedy_bn_seq.astype(i32).reshape(-1)
    rvi = greedy_vis_seq.astype(i32).reshape(-1)
    gt = gt_gts.astype(i32).reshape(-1)
    ncap_i = ncap.astype(i32)
    st = jnp.pad(st2towidx.astype(i32), (0, ST_PAD - st2towidx.shape[0]))
    tok = jnp.pad(token_scores.astype(jnp.float32),
                  (0, TOK_PAD - token_scores.shape[0]))

    mesh = plsc.VectorSubcoreMesh(core_axis_name="c", subcore_axis_name="s",
                                  num_cores=2, num_subcores=16)
    sc = pl.kernel(
        _sc_body,
        out_type=(jax.ShapeDtypeStruct((BATCH * SEQ,), jnp.float32),
                  jax.ShapeDtypeStruct((NW, 16), jnp.float32)),
        mesh=mesh,
        compiler_params=pltpu.CompilerParams(needs_layout_passes=False),
        scratch_types=[
            pltpu.VMEM((TOK_PAD,), jnp.float32),
            pltpu.VMEM((ST_PAD,), i32),
            pltpu.VMEM((SEQ_ELEMS_W,), i32),
            pltpu.VMEM((SEQ_ELEMS_W,), i32),
            pltpu.VMEM((SEQ_ELEMS_W,), i32),
            pltpu.VMEM((SEQ_ELEMS_W,), i32),
            pltpu.VMEM((SEQ_ELEMS_W,), i32),
            pltpu.VMEM((SEQ_ELEMS_W,), i32),
            pltpu.VMEM((GT_ELEMS_W,), i32),
            pltpu.VMEM((IMGS_W,), i32),
            pltpu.VMEM((IMGS_W,), jnp.float32),
            pltpu.VMEM((SEQ_ELEMS_W,), jnp.float32),
            pltpu.VMEM((16,), jnp.float32),
            pltpu.SemaphoreType.DMA,
        ],
    )
    rew_flat, psum = sc(gtx, gbn, gvi, rtx, rbn, rvi, gt, ncap_i, st, tok)

    mean_arr = pl.pallas_call(
        _mean_body,
        out_shape=jax.ShapeDtypeStruct((1, 1), jnp.float32),
    )(psum)

    return rew_flat.reshape(BATCH, SEQ), mean_arr[0, 0]
